# baseline (speedup 1.0000x reference)
import jax
import jax.numpy as jnp
from jax.experimental import pallas as pl
from jax.experimental.pallas import tpu as pltpu


def _copy_body(x_ref, e_ref, xo_ref, eo_ref, sem_x, sem_e):
    cx = pltpu.make_async_copy(x_ref, xo_ref, sem_x)
    ce = pltpu.make_async_copy(e_ref, eo_ref, sem_e)
    cx.start()
    ce.start()
    cx.wait()
    ce.wait()


def kernel(x, edge_index, edge_attr):
    del edge_index
    x_out, e_out = pl.pallas_call(
        _copy_body,
        in_specs=[
            pl.BlockSpec(memory_space=pl.ANY),
            pl.BlockSpec(memory_space=pl.ANY),
        ],
        out_specs=[
            pl.BlockSpec(memory_space=pl.ANY),
            pl.BlockSpec(memory_space=pl.ANY),
        ],
        out_shape=[
            jax.ShapeDtypeStruct(x.shape, x.dtype),
            jax.ShapeDtypeStruct(edge_attr.shape, edge_attr.dtype),
        ],
        scratch_shapes=[pltpu.SemaphoreType.DMA, pltpu.SemaphoreType.DMA],
    )(x, edge_attr)
    return (x_out, e_out)


# speedup vs baseline: 17.3607x; 17.3607x over previous
import jax
import jax.numpy as jnp
from jax.experimental import pallas as pl
from jax.experimental.pallas import tpu as pltpu

_GRID = 10
_XB = 10000 // _GRID
_EB = 40000 // _GRID


def _copy_body(x_ref, e_ref, xo_ref, eo_ref):
    xo_ref[...] = x_ref[...]
    eo_ref[...] = e_ref[...]


def kernel(x, edge_index, edge_attr):
    del edge_index
    e2 = edge_attr.reshape(40000, 128)
    x_out, e_out = pl.pallas_call(
        _copy_body,
        grid=(_GRID,),
        in_specs=[
            pl.BlockSpec((_XB, 128), lambda i: (i, jnp.int32(0))),
            pl.BlockSpec((_EB, 128), lambda i: (i, jnp.int32(0))),
        ],
        out_specs=[
            pl.BlockSpec((_XB, 128), lambda i: (i, jnp.int32(0))),
            pl.BlockSpec((_EB, 128), lambda i: (i, jnp.int32(0))),
        ],
        out_shape=[
            jax.ShapeDtypeStruct((10000, 128), x.dtype),
            jax.ShapeDtypeStruct((40000, 128), edge_attr.dtype),
        ],
        compiler_params=pltpu.CompilerParams(
            dimension_semantics=("arbitrary",),
        ),
    )(x, e2)
    return (x_out, e_out.reshape(320000, 16))


# speedup vs baseline: 19.0514x; 1.0974x over previous
import jax
import jax.numpy as jnp
from jax.experimental import pallas as pl
from jax.experimental.pallas import tpu as pltpu

_GRID = 25
_XB = 10000 // _GRID
_EB = 320000 // _GRID


def _copy_body(x_ref, e_ref, xo_ref, eo_ref):
    xo_ref[...] = x_ref[...]
    eo_ref[...] = e_ref[...]


def kernel(x, edge_index, edge_attr):
    del edge_index
    x_out, e_out = pl.pallas_call(
        _copy_body,
        grid=(_GRID,),
        in_specs=[
            pl.BlockSpec((_XB, 128), lambda i: (i, jnp.int32(0))),
            pl.BlockSpec((_EB, 16), lambda i: (i, jnp.int32(0))),
        ],
        out_specs=[
            pl.BlockSpec((_XB, 128), lambda i: (i, jnp.int32(0))),
            pl.BlockSpec((_EB, 16), lambda i: (i, jnp.int32(0))),
        ],
        out_shape=[
            jax.ShapeDtypeStruct(x.shape, x.dtype),
            jax.ShapeDtypeStruct(edge_attr.shape, edge_attr.dtype),
        ],
        compiler_params=pltpu.CompilerParams(
            dimension_semantics=("arbitrary",),
        ),
    )(x, edge_attr)
    return (x_out, e_out)
